# baseline (speedup 1.0000x reference)
import functools

import jax
import jax.numpy as jnp
from jax import lax
from jax.experimental import pallas as pl
from jax.experimental.pallas import tpu as pltpu
from jax.experimental.pallas import tpu_sc as plsc

_COLS = (3, 17, 42, 77, 99, 110, 5, 63)
_SIGNS = (-1.0, 1.0, -1.0, 1.0, -1.0, 1.0, -1.0, 1.0)
_L = 16


def _make_sc_call(num_rows, num_cols, nc, ns, chunk_rows):
    nw = nc * ns
    rows_per_w = num_rows // nw
    n_chunks = rows_per_w // chunk_rows
    n_groups = chunk_rows // _L
    nlit = len(_COLS)

    mesh = plsc.VectorSubcoreMesh(
        core_axis_name="c", subcore_axis_name="s",
        num_cores=nc, num_subcores=ns)

    @functools.partial(
        pl.kernel,
        out_type=jax.ShapeDtypeStruct((num_rows, nlit), jnp.float32),
        mesh=mesh,
        compiler_params=pltpu.CompilerParams(needs_layout_passes=False),
        scratch_types=[
            pltpu.VMEM((chunk_rows, num_cols), jnp.float32),
            pltpu.VMEM((chunk_rows, num_cols), jnp.float32),
            pltpu.VMEM((chunk_rows, nlit), jnp.float32),
            pltpu.VMEM((_L,), jnp.float32),
            pltpu.SemaphoreType.DMA,
            pltpu.SemaphoreType.DMA,
            pltpu.SemaphoreType.DMA,
        ],
    )
    def sc_kernel(in_hbm, cw_hbm, out_hbm, in_v0, in_v1, out_v,
                  cw_v, si0, si1, so0):
        wid = lax.axis_index("s") * nc + lax.axis_index("c")
        base = wid * rows_per_w

        pltpu.sync_copy(cw_hbm, cw_v)
        w = cw_v[...]
        iota = lax.iota(jnp.int32, _L)
        col_idx = [jnp.full((_L,), c, jnp.int32) for c in _COLS]
        lit_idx = [jnp.full((_L,), l, jnp.int32) for l in range(nlit)]

        def make_group(in_v):
            def group(t, _):
                rows = t * _L + iota
                vals = [plsc.load_gather(in_v, [rows, col_idx[l]])
                        for l in range(nlit)]
                sv = [v if s > 0 else -v for v, s in zip(vals, _SIGNS)]
                m = sv[0]
                for x in sv[1:]:
                    m = jnp.maximum(m, x)
                e = [jnp.exp(x - m) for x in sv]
                tot = e[0]
                for x in e[1:]:
                    tot = tot + x
                r_pos = w / tot
                r_neg = -r_pos
                for l in range(nlit):
                    d = e[l] * (r_pos if _SIGNS[l] > 0 else r_neg)
                    plsc.store_scatter(out_v, [rows, lit_idx[l]], d)
                return 0
            return group

        def in_slice(g):
            return in_hbm.at[pl.ds(base + g * chunk_rows, chunk_rows), :]

        def out_slice(g):
            return out_hbm.at[pl.ds(base + g * chunk_rows, chunk_rows), :]

        in_bufs, in_sems = [in_v0, in_v1], [si0, si1]
        in_desc = [None, None]
        out_desc = None
        in_desc[0] = pltpu.async_copy(in_slice(0), in_bufs[0], in_sems[0])
        for g in range(n_chunks):
            b = g & 1
            if g + 1 < n_chunks:
                in_desc[1 - b] = pltpu.async_copy(
                    in_slice(g + 1), in_bufs[1 - b], in_sems[1 - b])
            in_desc[b].wait()
            if out_desc is not None:
                out_desc.wait()
            lax.fori_loop(0, n_groups, make_group(in_bufs[b]), 0)
            out_desc = pltpu.async_copy(out_v, out_slice(g), so0)
        out_desc.wait()

    return sc_kernel


def kernel(inputs, clause_weight):
    num_rows, num_cols = inputs.shape
    cw16 = jnp.broadcast_to(clause_weight.astype(jnp.float32), (_L,))
    sc = _make_sc_call(num_rows, num_cols, nc=2, ns=16, chunk_rows=256)
    delta = sc(inputs, cw16)
    scatter_literal_indices = jnp.array(_COLS, dtype=jnp.int32).reshape(-1, 1)
    return (delta, scatter_literal_indices)


# speedup vs baseline: 1.0101x; 1.0101x over previous
import functools

import jax
import jax.numpy as jnp
from jax import lax
from jax.experimental import pallas as pl
from jax.experimental.pallas import tpu as pltpu
from jax.experimental.pallas import tpu_sc as plsc

_COLS = (3, 17, 42, 77, 99, 110, 5, 63)
_SIGNS = (-1.0, 1.0, -1.0, 1.0, -1.0, 1.0, -1.0, 1.0)
_L = 16


def _make_sc_call(num_rows, num_cols, nc, ns, chunk_rows):
    nw = nc * ns
    rows_per_w = num_rows // nw
    n_chunks = rows_per_w // chunk_rows
    n_groups = chunk_rows // _L
    nlit = len(_COLS)

    mesh = plsc.VectorSubcoreMesh(
        core_axis_name="c", subcore_axis_name="s",
        num_cores=nc, num_subcores=ns)

    @functools.partial(
        pl.kernel,
        out_type=jax.ShapeDtypeStruct((num_rows, nlit), jnp.float32),
        mesh=mesh,
        compiler_params=pltpu.CompilerParams(needs_layout_passes=False),
        scratch_types=[
            pltpu.VMEM((chunk_rows * num_cols,), jnp.float32),
            pltpu.VMEM((chunk_rows * num_cols,), jnp.float32),
            pltpu.VMEM((chunk_rows, nlit), jnp.float32),
            pltpu.VMEM((_L,), jnp.float32),
            pltpu.SemaphoreType.DMA,
            pltpu.SemaphoreType.DMA,
            pltpu.SemaphoreType.DMA,
        ],
    )
    def sc_kernel(in_hbm, cw_hbm, out_hbm, in_v0, in_v1, out_v0,
                  cw_v, si0, si1, so0):
        wid = lax.axis_index("s") * nc + lax.axis_index("c")
        base = wid * rows_per_w

        pltpu.sync_copy(cw_hbm, cw_v)
        w = cw_v[...]
        iota = lax.iota(jnp.int32, _L)
        in_stride = iota * num_cols
        lit_idx = [jnp.full((_L,), l, jnp.int32) for l in range(nlit)]

        def make_group(in_v, out_v):
            def group(t, _):
                rows = t * _L + iota
                in_base = t * (_L * num_cols) + in_stride
                vals = [plsc.load_gather(in_v, [in_base + c]) for c in _COLS]
                sv = [v if s > 0 else -v for v, s in zip(vals, _SIGNS)]
                m = sv[0]
                for x in sv[1:]:
                    m = jnp.maximum(m, x)
                e = [jnp.exp(x - m) for x in sv]
                tot = e[0]
                for x in e[1:]:
                    tot = tot + x
                r_pos = w / tot
                r_neg = -r_pos
                for l in range(nlit):
                    d = e[l] * (r_pos if _SIGNS[l] > 0 else r_neg)
                    plsc.store_scatter(out_v, [rows, lit_idx[l]], d)
                return 0
            return group

        def in_slice(g):
            return in_hbm.at[
                pl.ds((base + g * chunk_rows) * num_cols,
                      chunk_rows * num_cols)]

        def out_slice(g):
            return out_hbm.at[pl.ds(base + g * chunk_rows, chunk_rows), :]

        in_bufs, in_sems = [in_v0, in_v1], [si0, si1]
        in_desc = [None, None]
        out_desc = None
        in_desc[0] = pltpu.async_copy(in_slice(0), in_bufs[0], in_sems[0])
        for g in range(n_chunks):
            b = g & 1
            if g + 1 < n_chunks:
                in_desc[1 - b] = pltpu.async_copy(
                    in_slice(g + 1), in_bufs[1 - b], in_sems[1 - b])
            in_desc[b].wait()
            if out_desc is not None:
                out_desc.wait()
            lax.fori_loop(0, n_groups, make_group(in_bufs[b], out_v0), 0)
            out_desc = pltpu.async_copy(out_v0, out_slice(g), so0)
        out_desc.wait()

    return sc_kernel


def kernel(inputs, clause_weight):
    num_rows, num_cols = inputs.shape
    cw16 = jnp.broadcast_to(clause_weight.astype(jnp.float32), (_L,))
    sc = _make_sc_call(num_rows, num_cols, nc=2, ns=16, chunk_rows=256)
    delta = sc(inputs.reshape(-1), cw16)
    scatter_literal_indices = jnp.array(_COLS, dtype=jnp.int32).reshape(-1, 1)
    return (delta, scatter_literal_indices)


# speedup vs baseline: 1.7531x; 1.7357x over previous
import functools

import jax
import jax.numpy as jnp
from jax import lax
from jax.experimental import pallas as pl
from jax.experimental.pallas import tpu as pltpu
from jax.experimental.pallas import tpu_sc as plsc

_COLS = (3, 17, 42, 77, 99, 110, 5, 63)
_SIGNS = (-1.0, 1.0, -1.0, 1.0, -1.0, 1.0, -1.0, 1.0)
_L = 16


def _make_sc_call(num_rows, num_cols, nc, ns, chunk_rows):
    nw = nc * ns
    rows_per_w = num_rows // nw
    n_chunks = rows_per_w // chunk_rows
    n_groups = chunk_rows // _L
    nlit = len(_COLS)

    mesh = plsc.VectorSubcoreMesh(
        core_axis_name="c", subcore_axis_name="s",
        num_cores=nc, num_subcores=ns)

    @functools.partial(
        pl.kernel,
        out_type=jax.ShapeDtypeStruct((nlit, num_rows), jnp.float32),
        mesh=mesh,
        compiler_params=pltpu.CompilerParams(needs_layout_passes=False),
        scratch_types=[
            pltpu.VMEM((chunk_rows, num_cols), jnp.float32),
            pltpu.VMEM((chunk_rows, num_cols), jnp.float32),
            pltpu.VMEM((nlit, chunk_rows), jnp.float32),
            pltpu.VMEM((_L,), jnp.float32),
            pltpu.SemaphoreType.DMA,
            pltpu.SemaphoreType.DMA,
            pltpu.SemaphoreType.DMA,
        ],
    )
    def sc_kernel(in_hbm, cw_hbm, out_hbm, in_v0, in_v1, out_v,
                  cw_v, si0, si1, so0):
        wid = lax.axis_index("s") * nc + lax.axis_index("c")
        base = wid * rows_per_w

        pltpu.sync_copy(cw_hbm, cw_v)
        w = cw_v[...]
        iota = lax.iota(jnp.int32, _L)
        col_idx = [jnp.full((_L,), c, jnp.int32) for c in _COLS]

        def make_group(in_v):
            def group(t, _):
                r0 = t * _L
                rows = r0 + iota
                vals = [plsc.load_gather(in_v, [rows, col_idx[l]])
                        for l in range(nlit)]
                sv = [v if s > 0 else -v for v, s in zip(vals, _SIGNS)]
                m = sv[0]
                for x in sv[1:]:
                    m = jnp.maximum(m, x)
                e = [jnp.exp(x - m) for x in sv]
                tot = e[0]
                for x in e[1:]:
                    tot = tot + x
                r_pos = w / tot
                r_neg = -r_pos
                for l in range(nlit):
                    out_v[l, pl.ds(r0, _L)] = \
                        e[l] * (r_pos if _SIGNS[l] > 0 else r_neg)
                return 0
            return group

        def in_slice(g):
            return in_hbm.at[pl.ds(base + g * chunk_rows, chunk_rows), :]

        def out_slice(g):
            return out_hbm.at[:, pl.ds(base + g * chunk_rows, chunk_rows)]

        in_bufs, in_sems = [in_v0, in_v1], [si0, si1]
        in_desc = [None, None]
        out_desc = None
        in_desc[0] = pltpu.async_copy(in_slice(0), in_bufs[0], in_sems[0])
        for g in range(n_chunks):
            b = g & 1
            if g + 1 < n_chunks:
                in_desc[1 - b] = pltpu.async_copy(
                    in_slice(g + 1), in_bufs[1 - b], in_sems[1 - b])
            in_desc[b].wait()
            if out_desc is not None:
                out_desc.wait()
            lax.fori_loop(0, n_groups, make_group(in_bufs[b]), 0)
            out_desc = pltpu.async_copy(out_v, out_slice(g), so0)
        out_desc.wait()

    return sc_kernel


def kernel(inputs, clause_weight):
    num_rows, num_cols = inputs.shape
    cw16 = jnp.broadcast_to(clause_weight.astype(jnp.float32), (_L,))
    sc = _make_sc_call(num_rows, num_cols, nc=2, ns=16, chunk_rows=256)
    delta = sc(inputs, cw16).T
    scatter_literal_indices = jnp.array(_COLS, dtype=jnp.int32).reshape(-1, 1)
    return (delta, scatter_literal_indices)
